# Initial kernel scaffold; baseline (speedup 1.0000x reference)
import jax
import jax.numpy as jnp
from jax.experimental import pallas as pl


def kernel(x, edge_index, W, b):
    raise NotImplementedError("write your pallas kernel here")


# speedup vs baseline: 7.0697x; 7.0697x over previous
import functools

import jax
import jax.numpy as jnp
from jax import lax
from jax.experimental import pallas as pl
from jax.experimental.pallas import tpu as pltpu
from jax.experimental.pallas import tpu_sc as plsc

N = 10000
E = 320000
D = 128
DP = 144
NC = 2
NS = 16
NW = NC * NS
C = 80
NCHUNK = E // (NW * C)
ROWS_PER_TILE = N // NS
ZROWS = 25
OROWS = 125

_mesh = plsc.VectorSubcoreMesh(
    core_axis_name="c", subcore_axis_name="s", num_cores=NC, num_subcores=NS
)


@functools.partial(
    pl.kernel,
    out_type=jax.ShapeDtypeStruct((NC, N, DP), jnp.float32),
    mesh=_mesh,
    scratch_types=[
        pltpu.VMEM((NCHUNK, C), jnp.int32),
        pltpu.VMEM((NCHUNK, C), jnp.int32),
        pltpu.VMEM((C, DP), jnp.float32),
        pltpu.VMEM((ZROWS, DP), jnp.float32),
        pltpu.VMEM_SHARED((N, DP), jnp.float32),
        pltpu.SemaphoreType.DMA,
    ],
    compiler_params=pltpu.CompilerParams(use_tc_tiling_on_sc=False),
)
def _sc_aggregate(xp_hbm, src_hbm, dst_hbm, out_hbm,
                  src_v, dst_v, rows_v, zbuf, acc, sem):
    cid = lax.axis_index("c")
    sid = lax.axis_index("s")
    wid = cid * NS + sid

    def _zero_row(i, _):
        r = i // (DP // 16)
        f = i % (DP // 16)
        zbuf[r, pl.ds(f * 16, 16)] = jnp.zeros((16,), jnp.float32)
        return 0
    lax.fori_loop(0, ZROWS * (DP // 16), _zero_row, 0)
    for kz in range(ROWS_PER_TILE // ZROWS):
        pltpu.sync_copy(zbuf, acc.at[pl.ds(sid * ROWS_PER_TILE + kz * ZROWS, ZROWS)])
    plsc.subcore_barrier()

    base = wid * NCHUNK
    pltpu.sync_copy(src_hbm.at[pl.ds(base, NCHUNK)], src_v)
    pltpu.sync_copy(dst_hbm.at[pl.ds(base, NCHUNK)], dst_v)

    def _edge_chunk(j, _):
        pltpu.async_copy(xp_hbm.at[src_v.at[j]], rows_v, sem).wait()
        pltpu.sync_copy(rows_v, acc.at[dst_v.at[j]], add=True)
        return 0
    lax.fori_loop(0, NCHUNK, _edge_chunk, 0)

    plsc.subcore_barrier()
    for kz in range(ROWS_PER_TILE // OROWS):
        r0 = sid * ROWS_PER_TILE + kz * OROWS
        pltpu.sync_copy(acc.at[pl.ds(r0, OROWS)], out_hbm.at[cid].at[pl.ds(r0, OROWS)])


def _tc_finish(p_ref, x_ref, w_ref, b_ref, o_ref):
    p = p_ref[0] + p_ref[1]
    agg = p[:, :D]
    deg = jnp.maximum(p[:, D:D + 1], 1.0)
    h = agg / deg + x_ref[...]
    y = jnp.dot(h, w_ref[...], preferred_element_type=jnp.float32) + b_ref[...]
    o_ref[...] = jnp.maximum(y, 0.0)


def kernel(x, edge_index, W, b):
    ei = edge_index.astype(jnp.int32)
    src2d = ei[0].reshape(E // C, C)
    dst2d = ei[1].reshape(E // C, C)
    xp = jnp.concatenate(
        [x, jnp.ones((N, 1), x.dtype), jnp.zeros((N, DP - D - 1), x.dtype)], axis=1
    )
    partials = _sc_aggregate(xp, src2d, dst2d)

    BN = 1000
    out = pl.pallas_call(
        _tc_finish,
        grid=(N // BN,),
        in_specs=[
            pl.BlockSpec((NC, BN, DP), lambda i: (0, i, 0)),
            pl.BlockSpec((BN, D), lambda i: (i, 0)),
            pl.BlockSpec((D, D), lambda i: (0, 0)),
            pl.BlockSpec((1, D), lambda i: (0, 0)),
        ],
        out_specs=pl.BlockSpec((BN, D), lambda i: (i, 0)),
        out_shape=jax.ShapeDtypeStruct((N, D), jnp.float32),
    )(partials, x, W, b.reshape(1, D))
    return out


# speedup vs baseline: 8.6349x; 1.2214x over previous
import functools

import jax
import jax.numpy as jnp
from jax import lax
from jax.experimental import pallas as pl
from jax.experimental.pallas import tpu as pltpu
from jax.experimental.pallas import tpu_sc as plsc

N = 10000
E = 320000
D = 128
DP = 144
NC = 2
NS = 16
NW = NC * NS
C = 80
NCHUNK = E // (NW * C)
ROWS_PER_TILE = N // NS
ZROWS = 25
OROWS = 125
IH = 63

_mesh = plsc.VectorSubcoreMesh(
    core_axis_name="c", subcore_axis_name="s", num_cores=NC, num_subcores=NS
)


@functools.partial(
    pl.kernel,
    out_type=jax.ShapeDtypeStruct((NC, N, DP), jnp.float32),
    mesh=_mesh,
    scratch_types=[
        pltpu.VMEM((IH, C), jnp.int32),
        pltpu.VMEM((IH, C), jnp.int32),
        pltpu.VMEM((2, C, DP), jnp.float32),
        pltpu.VMEM((ZROWS, DP), jnp.float32),
        pltpu.VMEM_SHARED((N, DP), jnp.float32),
        pltpu.SemaphoreType.DMA((2,)),
    ],
    compiler_params=pltpu.CompilerParams(use_tc_tiling_on_sc=False),
)
def _sc_aggregate(xp_hbm, src_hbm, dst_hbm, out_hbm,
                  src_v, dst_v, rows_v, zbuf, acc, sem):
    cid = lax.axis_index("c")
    sid = lax.axis_index("s")
    wid = cid * NS + sid

    def _zero_row(i, _):
        r = i // (DP // 16)
        f = i % (DP // 16)
        zbuf[r, pl.ds(f * 16, 16)] = jnp.zeros((16,), jnp.float32)
        return 0
    lax.fori_loop(0, ZROWS * (DP // 16), _zero_row, 0)
    for kz in range(ROWS_PER_TILE // ZROWS):
        pltpu.sync_copy(zbuf, acc.at[pl.ds(sid * ROWS_PER_TILE + kz * ZROWS, ZROWS)])
    plsc.subcore_barrier()

    pltpu.sync_copy(src_hbm.at[wid].at[pl.ds(0, IH)], src_v)
    pltpu.sync_copy(dst_hbm.at[wid].at[pl.ds(0, IH)], dst_v)

    def _gather(j, buf):
        row = jnp.where(j < IH, j, j - IH)
        pltpu.async_copy(xp_hbm.at[src_v.at[row]], rows_v.at[buf], sem.at[buf])

    def _gather_wait(buf):
        pltpu.make_async_copy(
            xp_hbm.at[src_v.at[0]], rows_v.at[buf], sem.at[buf]).wait()

    _gather(0, 0)

    def _edge_chunk(j, _):
        buf = lax.rem(j, 2)
        nbuf = 1 - buf
        _gather_wait(buf)

        @pl.when(j == IH - 1)
        def _():
            pltpu.sync_copy(src_hbm.at[wid].at[pl.ds(IH, NCHUNK - IH)],
                            src_v.at[pl.ds(0, NCHUNK - IH)])

        @pl.when(j + 1 < NCHUNK)
        def _():
            _gather(j + 1, nbuf)

        @pl.when(j == IH)
        def _():
            pltpu.sync_copy(dst_hbm.at[wid].at[pl.ds(IH, NCHUNK - IH)],
                            dst_v.at[pl.ds(0, NCHUNK - IH)])

        row = jnp.where(j < IH, j, j - IH)
        pltpu.sync_copy(rows_v.at[buf], acc.at[dst_v.at[row]], add=True)
        return 0
    lax.fori_loop(0, NCHUNK, _edge_chunk, 0)

    plsc.subcore_barrier()
    for kz in range(ROWS_PER_TILE // OROWS):
        r0 = sid * ROWS_PER_TILE + kz * OROWS
        pltpu.sync_copy(acc.at[pl.ds(r0, OROWS)], out_hbm.at[cid].at[pl.ds(r0, OROWS)])


def _tc_finish(p_ref, x_ref, w_ref, b_ref, o_ref):
    p = p_ref[0] + p_ref[1]
    agg = p[:, :D]
    deg = jnp.maximum(p[:, D:D + 1], 1.0)
    h = agg / deg + x_ref[...]
    y = jnp.dot(h, w_ref[...], preferred_element_type=jnp.float32) + b_ref[...]
    o_ref[...] = jnp.maximum(y, 0.0)


def kernel(x, edge_index, W, b):
    ei = edge_index.astype(jnp.int32)
    src2d = ei[0].reshape(NW, NCHUNK, C)
    dst2d = ei[1].reshape(NW, NCHUNK, C)
    xp = jnp.concatenate(
        [x, jnp.ones((N, 1), x.dtype), jnp.zeros((N, DP - D - 1), x.dtype)], axis=1
    )
    partials = _sc_aggregate(xp, src2d, dst2d)

    BN = 1000
    out = pl.pallas_call(
        _tc_finish,
        grid=(N // BN,),
        in_specs=[
            pl.BlockSpec((NC, BN, DP), lambda i: (0, i, 0)),
            pl.BlockSpec((BN, D), lambda i: (i, 0)),
            pl.BlockSpec((D, D), lambda i: (0, 0)),
            pl.BlockSpec((1, D), lambda i: (0, 0)),
        ],
        out_specs=pl.BlockSpec((BN, D), lambda i: (i, 0)),
        out_shape=jax.ShapeDtypeStruct((N, D), jnp.float32),
    )(partials, x, W, b.reshape(1, D))
    return out
